# baseline (speedup 1.0000x reference)
import functools

import jax
import jax.numpy as jnp
import numpy as np
from jax import lax
from jax.experimental import pallas as pl
from jax.experimental.pallas import tpu as pltpu
from jax.experimental.pallas import tpu_sc as plsc

C = 2048
E = 131072
D = 32
H = 32
K = 2
MSG_H = 64
MSG_O = 32
TAU = 0.1

_NC = 2
_NS = 16
_NW = _NC * _NS


def _make_gather(num_tables, chunk=512):
    per_w = E // _NW
    n_chunks = per_w // chunk
    mesh = plsc.VectorSubcoreMesh(core_axis_name="c", subcore_axis_name="s")
    out_type = [jax.ShapeDtypeStruct((E, D), jnp.float32)] * (2 * num_tables)
    scratch = [pltpu.VMEM((chunk,), jnp.int32), pltpu.VMEM((chunk,), jnp.int32)]
    scratch += [pltpu.VMEM((chunk, D), jnp.float32) for _ in range(2 * num_tables)]
    scratch += [pltpu.SemaphoreType.DMA]

    @functools.partial(pl.kernel, out_type=out_type, mesh=mesh,
                       scratch_types=scratch,
                       compiler_params=pltpu.CompilerParams(
                           use_tc_tiling_on_sc=False))
    def gather_kernel(*refs):
        tables = refs[:num_tables]
        send, rec = refs[num_tables], refs[num_tables + 1]
        outs = refs[num_tables + 2:3 * num_tables + 2]
        sidx, ridx = refs[3 * num_tables + 2], refs[3 * num_tables + 3]
        bufs = refs[3 * num_tables + 4:5 * num_tables + 4]
        sem = refs[5 * num_tables + 4]
        wid = lax.axis_index("s") * _NC + lax.axis_index("c")
        base = wid * per_w

        def step(t, carry):
            off = base + t * chunk
            pltpu.sync_copy(send.at[pl.ds(off, chunk)], sidx)
            pltpu.sync_copy(rec.at[pl.ds(off, chunk)], ridx)
            for j in range(num_tables):
                pltpu.async_copy(tables[j].at[sidx], bufs[2 * j], sem).wait()
                pltpu.async_copy(tables[j].at[ridx], bufs[2 * j + 1], sem).wait()
            for j in range(num_tables):
                pltpu.sync_copy(bufs[2 * j], outs[2 * j].at[pl.ds(off, chunk)])
                pltpu.sync_copy(bufs[2 * j + 1], outs[2 * j + 1].at[pl.ds(off, chunk)])
            return carry

        lax.fori_loop(0, n_chunks, step, 0)

    return gather_kernel


_gather2 = _make_gather(2)
_gather1 = _make_gather(1)


def _mlp(x, p, name):
    x = jax.nn.relu(x @ p[name + '_w1'] + p[name + '_b1'])
    x = jax.nn.relu(x @ p[name + '_w2'] + p[name + '_b2'])
    mean = jnp.mean(x, axis=0, keepdims=True)
    var = jnp.var(x, axis=0, keepdims=True)
    x = (x - mean) / jnp.sqrt(var + 1e-5)
    return x * p[name + '_g'] + p[name + '_be']


def _head_kernel(agg_ref, w1_ref, b1_ref, w2_ref, b2_ref, out_ref):
    pred = jnp.maximum(agg_ref[...] @ w1_ref[...] + b1_ref[...], 0.0)
    out_ref[...] = pred @ w2_ref[...] + b2_ref[...]


def kernel(data, params, send_idx, rec_idx):
    p = params
    x1 = _mlp(data, p, 'enc1')
    xs1, xr1, ds0, dr0 = _gather2(x1, data, send_idx, rec_idx)
    x = jnp.concatenate([xs1, xr1], axis=-1)
    x = _mlp(x, p, 'enc2')
    x_skip = x
    x = jax.ops.segment_sum(x, rec_idx, num_segments=C) / C
    x3 = _mlp(x, p, 'enc3')
    xs3, xr3 = _gather1(x3, send_idx, rec_idx)
    x = jnp.concatenate([xs3, xr3, x_skip], axis=-1)
    x = _mlp(x, p, 'enc4')
    logits = x @ p['fc_out_w'] + p['fc_out_b']
    u = jax.random.uniform(jax.random.key(42), logits.shape, minval=1e-6, maxval=1.0 - 1e-6)
    g = -jnp.log(-jnp.log(u))
    edges = jax.nn.softmax((logits + g) / TAU, axis=-1)
    prob = jax.nn.softmax(logits, axis=-1)

    pre_msg = jnp.concatenate([ds0, dr0], axis=-1)
    all_msgs = jnp.zeros((E, MSG_O), jnp.float32)
    for i in range(K):
        m = jax.nn.relu(pre_msg @ p['msg1_%d_w' % i] + p['msg1_%d_b' % i])
        m = jax.nn.relu(m @ p['msg2_%d_w' % i] + p['msg2_%d_b' % i])
        all_msgs = all_msgs + m * edges[:, i:i + 1]
    agg = jax.ops.segment_sum(all_msgs, rec_idx, num_segments=C) / C

    output = pl.pallas_call(
        _head_kernel,
        out_shape=jax.ShapeDtypeStruct((C, D), jnp.float32),
    )(agg, p['out1_w'], p['out1_b'], p['out2_w'], p['out2_b'])

    cell = send_idx * C + rec_idx
    eid = jnp.arange(E, dtype=jnp.int32) + 1
    win = jnp.zeros((C * C,), jnp.int32).at[cell].max(eid)
    src = jnp.clip(win - 1, 0, E - 1)
    mask = (win > 0).astype(jnp.float32)
    g0 = jnp.take(edges[:, 0], src) * mask
    g1 = jnp.take(edges[:, 1], src) * mask
    graphs = jnp.stack([g0.reshape(C, C), g1.reshape(C, C)])
    return graphs, output, prob
